# baseline (speedup 1.0000x reference)
import jax
import jax.numpy as jnp
from jax.experimental import pallas as pl
from jax.experimental.pallas import tpu as pltpu

_T = 16384
_D = 1024
_DO = 1024
_E = 64
_K = 8
_SCALE = 2.0
_TB = 2048


def _fused_kernel(x_ref, lg_ref, w_ref, a_ref, b_ref, o_ref):
    x = x_ref[:]
    lt = jnp.transpose(lg_ref[:])

    hx_t = jax.lax.dot_general(
        a_ref[:], x, (((1,), (1,)), ((), ())),
        preferred_element_type=jnp.float32,
        precision=jax.lax.Precision.DEFAULT)

    m = jnp.max(lt, axis=0, keepdims=True)
    e = jnp.exp(lt - m)
    s = e / jnp.sum(e, axis=0, keepdims=True)

    work = lt
    for _ in range(_K - 1):
        row_max = jnp.max(work, axis=0, keepdims=True)
        work = jnp.where(work == row_max, -jnp.inf, work)
    t_k = jnp.max(work, axis=0, keepdims=True)

    g_t = jnp.where(lt >= t_k, s, 0.0)
    ge_t = g_t * hx_t

    lora = jax.lax.dot_general(
        ge_t, b_ref[:], (((0,), (0,)), ((), ())),
        preferred_element_type=jnp.float32,
        precision=jax.lax.Precision.DEFAULT)

    base = jax.lax.dot_general(
        x, w_ref[:], (((1,), (1,)), ((), ())),
        preferred_element_type=jnp.float32,
        precision=jax.lax.Precision.DEFAULT)

    o_ref[:] = base + _SCALE * lora


@jax.jit
def kernel(x, router_logits, W, lora_A, lora_B):
    grid = (_T // _TB,)
    return pl.pallas_call(
        _fused_kernel,
        grid=grid,
        in_specs=[
            pl.BlockSpec((_TB, _D), lambda i: (i, 0)),
            pl.BlockSpec((_TB, _E), lambda i: (i, 0)),
            pl.BlockSpec((_DO, _D), lambda i: (0, 0)),
            pl.BlockSpec((_E, _D), lambda i: (0, 0)),
            pl.BlockSpec((_E, _DO), lambda i: (0, 0)),
        ],
        out_specs=pl.BlockSpec((_TB, _DO), lambda i: (i, 0)),
        out_shape=jax.ShapeDtypeStruct((_T, _DO), jnp.float32),
        compiler_params=pltpu.CompilerParams(
            dimension_semantics=("parallel",)),
    )(x, router_logits, W, lora_A, lora_B)


# speedup vs baseline: 1.1127x; 1.1127x over previous
import jax
import jax.numpy as jnp
from jax.experimental import pallas as pl
from jax.experimental.pallas import tpu as pltpu

_T = 16384
_D = 1024
_DO = 1024
_E = 64
_K = 8
_SCALE = 2.0
_TB = 2048


def _fused_kernel(x_ref, lt_ref, w_ref, a_ref, b_ref, o_ref):
    x = x_ref[:]
    lt = lt_ref[:]

    hx_t = jax.lax.dot_general(
        a_ref[:], x, (((1,), (1,)), ((), ())),
        preferred_element_type=jnp.float32,
        precision=jax.lax.Precision.DEFAULT)

    m = jnp.max(lt, axis=0, keepdims=True)
    e = jnp.exp(lt - m)
    s = e / jnp.sum(e, axis=0, keepdims=True)

    work = lt
    for _ in range(_K - 1):
        row_max = jnp.max(work, axis=0, keepdims=True)
        work = jnp.where(work == row_max, -jnp.inf, work)
    t_k = jnp.max(work, axis=0, keepdims=True)

    g_t = jnp.where(lt >= t_k, s, 0.0)
    ge_t = g_t * hx_t

    lora = jax.lax.dot_general(
        ge_t, b_ref[:], (((0,), (0,)), ((), ())),
        preferred_element_type=jnp.float32,
        precision=jax.lax.Precision.DEFAULT)

    base = jax.lax.dot_general(
        x, w_ref[:], (((1,), (1,)), ((), ())),
        preferred_element_type=jnp.float32,
        precision=jax.lax.Precision.DEFAULT)

    o_ref[:] = base + _SCALE * lora


@jax.jit
def kernel(x, router_logits, W, lora_A, lora_B):
    lt = router_logits.T
    grid = (_T // _TB,)
    return pl.pallas_call(
        _fused_kernel,
        grid=grid,
        in_specs=[
            pl.BlockSpec((_TB, _D), lambda i: (i, 0)),
            pl.BlockSpec((_E, _TB), lambda i: (0, i)),
            pl.BlockSpec((_DO, _D), lambda i: (0, 0)),
            pl.BlockSpec((_E, _D), lambda i: (0, 0)),
            pl.BlockSpec((_E, _DO), lambda i: (0, 0)),
        ],
        out_specs=pl.BlockSpec((_TB, _DO), lambda i: (i, 0)),
        out_shape=jax.ShapeDtypeStruct((_T, _DO), jnp.float32),
        compiler_params=pltpu.CompilerParams(
            dimension_semantics=("parallel",)),
    )(x, lt, W, lora_A, lora_B)


# speedup vs baseline: 1.1132x; 1.0005x over previous
import jax
import jax.numpy as jnp
from jax.experimental import pallas as pl
from jax.experimental.pallas import tpu as pltpu

_T = 16384
_D = 1024
_DO = 1024
_E = 64
_K = 8
_SCALE = 2.0
_TB = 2048


def _fused_kernel(x_ref, lt_ref, w_ref, a_ref, b_ref, o_ref):
    x = x_ref[:]
    lt = lt_ref[:]

    hx_t = jax.lax.dot_general(
        a_ref[:], x, (((1,), (1,)), ((), ())),
        preferred_element_type=jnp.float32,
        precision=jax.lax.Precision.DEFAULT)

    m = jnp.max(lt, axis=0, keepdims=True)
    e = jnp.exp(lt - m)
    s = e / jnp.sum(e, axis=0, keepdims=True)

    work = lt
    for _ in range(_K - 1):
        row_max = jnp.max(work, axis=0, keepdims=True)
        work = jnp.where(work == row_max, -jnp.inf, work)
    t_k = jnp.max(work, axis=0, keepdims=True)

    g_t = jnp.where(lt >= t_k, s, 0.0)
    ge_t = g_t * hx_t

    lora = jax.lax.dot_general(
        ge_t, b_ref[:], (((0,), (0,)), ((), ())),
        preferred_element_type=jnp.float32,
        precision=jax.lax.Precision.DEFAULT)

    base = jax.lax.dot_general(
        x, w_ref[:], (((1,), (1,)), ((), ())),
        preferred_element_type=jnp.float32,
        precision=jax.lax.Precision.DEFAULT)

    o_ref[:] = base + _SCALE * lora


@jax.jit
def kernel(x, router_logits, W, lora_A, lora_B):
    lt = router_logits.T
    grid = (_T // _TB,)
    return pl.pallas_call(
        _fused_kernel,
        grid=grid,
        in_specs=[
            pl.BlockSpec((_TB, _D), lambda i: (i, 0)),
            pl.BlockSpec((_E, _TB), lambda i: (0, i)),
            pl.BlockSpec((_DO, _D), lambda i: (0, 0)),
            pl.BlockSpec((_E, _D), lambda i: (0, 0)),
            pl.BlockSpec((_E, _DO), lambda i: (0, 0)),
        ],
        out_specs=pl.BlockSpec((_TB, _DO), lambda i: (i, 0)),
        out_shape=jax.ShapeDtypeStruct((_T, _DO), jnp.float32),
        compiler_params=pltpu.CompilerParams(
            dimension_semantics=("parallel",),
            allow_input_fusion=[False, True, False, False, False]),
    )(x, lt, W, lora_A, lora_B)


# speedup vs baseline: 1.1139x; 1.0006x over previous
import jax
import jax.numpy as jnp
from jax.experimental import pallas as pl
from jax.experimental.pallas import tpu as pltpu

_T = 16384
_D = 1024
_DO = 1024
_E = 64
_K = 8
_SCALE = 2.0
_TB = 2048


def _fused_kernel(x_ref, lt_ref, w_ref, a_ref, b_ref, o_ref):
    x = x_ref[:]
    lt = lt_ref[:]

    hx_t = jax.lax.dot_general(
        a_ref[:], x, (((1,), (1,)), ((), ())),
        preferred_element_type=jnp.float32,
        precision=jax.lax.Precision.DEFAULT)

    m = jnp.max(lt, axis=0, keepdims=True)
    e = jnp.exp(lt - m)
    s = e / jnp.sum(e, axis=0, keepdims=True)

    work = lt
    for _ in range(_K - 1):
        row_max = jnp.max(work, axis=0, keepdims=True)
        work = jnp.where(work == row_max, -jnp.inf, work)
    t_k = jnp.max(work, axis=0, keepdims=True)

    g_t = jnp.where(lt >= t_k, s, 0.0)
    ge_t = g_t * hx_t

    lora = jax.lax.dot_general(
        ge_t, b_ref[:], (((0,), (0,)), ((), ())),
        preferred_element_type=jnp.float32,
        precision=jax.lax.Precision.DEFAULT)

    base = jax.lax.dot_general(
        x, w_ref[:], (((1,), (1,)), ((), ())),
        preferred_element_type=jnp.float32,
        precision=jax.lax.Precision.DEFAULT)

    o_ref[:] = base + _SCALE * lora


@jax.jit
def kernel(x, router_logits, W, lora_A, lora_B):
    lt = router_logits.T
    grid = (_T // _TB,)
    return pl.pallas_call(
        _fused_kernel,
        grid=grid,
        in_specs=[
            pl.BlockSpec((_TB, _D), lambda i: (i, 0)),
            pl.BlockSpec((_E, _TB), lambda i: (0, i)),
            pl.BlockSpec((_DO, _D), lambda i: (0, 0)),
            pl.BlockSpec((_E, _D), lambda i: (0, 0)),
            pl.BlockSpec((_E, _DO), lambda i: (0, 0)),
        ],
        out_specs=pl.BlockSpec((_TB, _DO), lambda i: (i, 0)),
        out_shape=jax.ShapeDtypeStruct((_T, _DO), jnp.float32),
        compiler_params=pltpu.CompilerParams(
            dimension_semantics=("parallel",)),
    )(x, lt, W, lora_A, lora_B)
